# baseline (speedup 1.0000x reference)
import functools

import jax
import jax.numpy as jnp
from jax import lax
from jax.experimental import pallas as pl
from jax.experimental.pallas import tpu as pltpu
from jax.experimental.pallas import tpu_sc as plsc

N = 50000
E = 800000
F = 32
EPS = 1e-5
SLOPE = 0.1

NCORE = 2
NSUB = 16
CH = 256
SB = 128
NSC = CH // SB
NBUF = 3
EP = 811008
NP = 50048
ROWS_PER_SUB = NP // NSUB


def _sc_spmm(table, src2, dst_r, w_p, per_core_edges):
  nchunks = (EP // NSUB) // CH
  ntrip = nchunks // NBUF
  mesh = plsc.VectorSubcoreMesh(core_axis_name="c", subcore_axis_name="s")
  zeros = jnp.zeros((NP, F), jnp.float32)

  @functools.partial(
      pl.kernel,
      out_type=jax.ShapeDtypeStruct((NCORE, NP, F), jnp.float32),
      mesh=mesh,
      compiler_params=pltpu.CompilerParams(use_tc_tiling_on_sc=False),
      scratch_types=[
          pltpu.VMEM((NBUF, CH), jnp.int32),
          pltpu.VMEM((NBUF, NSC, SB), jnp.int32),
          pltpu.VMEM((NBUF, CH), jnp.float32),
          pltpu.VMEM((NBUF, CH, F), jnp.float32),
          pltpu.VMEM_SHARED((NP, F), jnp.float32),
          pltpu.SemaphoreType.DMA((NBUF,)),
          pltpu.SemaphoreType.DMA((NBUF,)),
          pltpu.SemaphoreType.DMA((NBUF,)),
      ],
  )
  def spmm(table_h, src_h, dstr_h, w_h, zeros_h, out_h,
           srcv, dstv, wv, rows, acc, gsem, ssem, lsem):
    c = lax.axis_index("c")
    s = lax.axis_index("s")

    r0 = pl.multiple_of(s * ROWS_PER_SUB, 8)
    pltpu.sync_copy(zeros_h.at[pl.ds(r0, ROWS_PER_SUB)],
                    acc.at[pl.ds(r0, ROWS_PER_SUB)])
    plsc.subcore_barrier()

    base = s * (EP // NSUB)
    wbase = c * EP if per_core_edges else 0
    dbase = c * (EP // SB) if per_core_edges else 0

    def issue_linear_sw(k, b):
      e0 = pl.multiple_of(base + k * CH, CH)
      pltpu.async_copy(src_h.at[pl.ds(pl.multiple_of(c * EP + e0, CH), CH)],
                       srcv.at[b], lsem.at[b])
      pltpu.async_copy(w_h.at[pl.ds(pl.multiple_of(wbase + e0, CH), CH)],
                       wv.at[b], lsem.at[b])

    def issue_linear_d(k, b):
      e0 = pl.multiple_of(base + k * CH, CH)
      pltpu.async_copy(
          dstr_h.at[pl.ds(pl.multiple_of(dbase + e0 // SB, NSC), NSC)],
          dstv.at[b], lsem.at[b])

    def wait_linear(b):
      pltpu.make_async_copy(src_h.at[pl.ds(0, CH)], srcv.at[b],
                            lsem.at[b]).wait()
      pltpu.make_async_copy(w_h.at[pl.ds(0, CH)], wv.at[b],
                            lsem.at[b]).wait()
      pltpu.make_async_copy(dstr_h.at[pl.ds(0, NSC)], dstv.at[b],
                            lsem.at[b]).wait()

    def issue_gather(b):
      pltpu.async_copy(table_h.at[srcv.at[b]], rows.at[b], gsem.at[b])

    def wait_gather(b):
      pltpu.make_async_copy(zeros_h.at[pl.ds(0, CH)], rows.at[b],
                            gsem.at[b]).wait()

    def multiply(b):
      def group_body(g, _):
        w16 = wv[b, pl.ds(g * 16, 16)]
        for u in range(16):
          e = g * 16 + u
          ws = w16[u]
          for h in range(0, F, 16):
            rows[b, e, pl.ds(h, 16)] = rows[b, e, pl.ds(h, 16)] * ws
        return 0
      lax.fori_loop(0, CH // 16, group_body, 0)

    def issue_scatters(b):
      for j in range(NSC):
        pltpu.async_copy(rows.at[b, pl.ds(j * SB, SB)],
                         acc.at[dstv.at[b, j]], ssem.at[b], add=True)

    def drain_scatters(b):
      for j in range(NSC):
        pltpu.make_async_copy(zeros_h.at[pl.ds(0, SB)],
                              rows.at[b, pl.ds(j * SB, SB)],
                              ssem.at[b]).wait()

    for b in range(NBUF - 1):
      issue_linear_sw(b, b)
      issue_linear_d(b, b)
      wait_linear(b)
      issue_gather(b)

    def trip_body(t, _):
      for b in range(NBUF):
        k = t * NBUF + b
        bp = (b + NBUF - 1) % NBUF
        q = k + NBUF - 1

        if b == 0:
          issue_linear_sw(q, bp)
        else:
          @pl.when(t < ntrip - 1)
          def _():
            issue_linear_sw(q, bp)

        wait_gather(b)
        multiply(b)
        issue_scatters(b)

        def finish_refill():
          drain_scatters(bp)
          issue_linear_d(q, bp)
          wait_linear(bp)
          issue_gather(bp)

        if b == 0:
          @pl.when(t == 0)
          def _():
            issue_linear_d(q, bp)
            wait_linear(bp)
            issue_gather(bp)

          @pl.when(t > 0)
          def _():
            finish_refill()
        else:
          @pl.when(t < ntrip - 1)
          def _():
            finish_refill()

          @pl.when(t == ntrip - 1)
          def _():
            drain_scatters(bp)
      return 0

    lax.fori_loop(0, ntrip, trip_body, 0)
    drain_scatters((nchunks - 1) % NBUF)

    plsc.subcore_barrier()
    pltpu.sync_copy(acc.at[pl.ds(r0, ROWS_PER_SUB)],
                    out_h.at[c, pl.ds(r0, ROWS_PER_SUB)])

  return spmm(table, src2, dst_r, w_p, zeros)


BN = 2000


def _tc_mm_body(x_ref, s2_ref, a_ref, b2_ref, bias_ref, y_ref, s1_ref, sq_ref):
  y = jnp.dot(x_ref[...], a_ref[...], preferred_element_type=jnp.float32)
  y -= jnp.dot(s2_ref[0], b2_ref[0], preferred_element_type=jnp.float32)
  y -= jnp.dot(s2_ref[1], b2_ref[1], preferred_element_type=jnp.float32)
  y += bias_ref[...]
  y_ref[...] = y
  part = jnp.sum(y.reshape(BN // 8, 8, F), axis=0)
  psq = jnp.sum((y * y).reshape(BN // 8, 8, F), axis=0)

  @pl.when(pl.program_id(0) == 0)
  def _():
    s1_ref[...] = part
    sq_ref[...] = psq

  @pl.when(pl.program_id(0) != 0)
  def _():
    s1_ref[...] += part
    sq_ref[...] += psq


def _tc_matmul(x, s2, a, b2, bias):
  d = x.shape[1]
  return pl.pallas_call(
      _tc_mm_body,
      grid=(N // BN,),
      in_specs=[
          pl.BlockSpec((BN, d), lambda i: (i, 0)),
          pl.BlockSpec((2, BN, F), lambda i: (0, i, 0)),
          pl.BlockSpec((d, F), lambda i: (0, 0)),
          pl.BlockSpec((2, F, F), lambda i: (0, 0, 0)),
          pl.BlockSpec((1, F), lambda i: (0, 0)),
      ],
      out_specs=[
          pl.BlockSpec((BN, F), lambda i: (i, 0)),
          pl.BlockSpec((8, F), lambda i: (0, 0)),
          pl.BlockSpec((8, F), lambda i: (0, 0)),
      ],
      out_shape=[
          jax.ShapeDtypeStruct((N, F), jnp.float32),
          jax.ShapeDtypeStruct((8, F), jnp.float32),
          jax.ShapeDtypeStruct((8, F), jnp.float32),
      ],
  )(x, s2, a, b2, bias)


def _tc_norm_body(y_ref, s1_ref, sq_ref, g_ref, b_ref, o_ref):
  s1 = jnp.sum(s1_ref[...], axis=0, keepdims=True)
  sq = jnp.sum(sq_ref[...], axis=0, keepdims=True)
  mean = s1 / N
  var = sq / N - mean * mean
  scale = g_ref[...] * lax.rsqrt(var + EPS)
  shift = b_ref[...] - mean * scale
  o = y_ref[...] * scale + shift
  o_ref[...] = jnp.where(o >= 0, o, SLOPE * o)


def _tc_norm(y, s1, sq, gamma, beta):
  return pl.pallas_call(
      _tc_norm_body,
      grid=(N // BN,),
      in_specs=[
          pl.BlockSpec((BN, F), lambda i: (i, 0)),
          pl.BlockSpec((8, F), lambda i: (0, 0)),
          pl.BlockSpec((8, F), lambda i: (0, 0)),
          pl.BlockSpec((1, F), lambda i: (0, 0)),
          pl.BlockSpec((1, F), lambda i: (0, 0)),
      ],
      out_specs=pl.BlockSpec((BN, F), lambda i: (i, 0)),
      out_shape=jax.ShapeDtypeStruct((N, F), jnp.float32),
  )(y, s1, sq, gamma, beta)


def _combine_weights(Ws):
  ks = jnp.arange(Ws.shape[0], dtype=jnp.float32)
  return jnp.sum(Ws, axis=0), jnp.einsum("k,kij->ij", ks, Ws)


def _tc_block(x, s2, Ws, bias, gamma, beta, b2):
  a, _ = _combine_weights(Ws)
  y, s1, sq = _tc_matmul(x, s2, a, b2, bias.reshape(1, F))
  return _tc_norm(y, s1, sq, gamma.reshape(1, F), beta.reshape(1, F))


def _prep_edges(ei, w):
  pad = EP - E
  src = jnp.concatenate([ei[0].astype(jnp.int32), jnp.zeros((pad,), jnp.int32)])
  dst = jnp.concatenate([ei[1].astype(jnp.int32), jnp.zeros((pad,), jnp.int32)])
  w_p = jnp.concatenate([w, jnp.zeros((pad,), jnp.float32)])
  return src, dst.reshape(EP // SB, SB), w_p


def kernel(x_t0, edge_weight_t, x_s0, edge_weight_s, Wt0, Wt1, bias_t, gamma_t,
           beta_t, Ws0, Ws1, bias_s, gamma_s, beta_s, edge_index_t,
           edge_index_s):
  src_t, dstr_t, wp_t = _prep_edges(edge_index_t, edge_weight_t)
  src_s, dstr_s, wp_s = _prep_edges(edge_index_s, edge_weight_s)
  zf = jnp.zeros((F, F), jnp.float32)

  def ch0(x0, src, dstr, wp, Ws, bias, gamma, beta):
    table = jnp.concatenate([x0[:, :F], x0[:, F:]], axis=0)
    src2 = jnp.concatenate([src, src + N])
    s2 = _sc_spmm(table, src2, dstr, wp, per_core_edges=False)[:, :N, :]
    _, b = _combine_weights(Ws)
    return _tc_block(x0, s2, Ws, bias[0], gamma[0], beta[0],
                     jnp.stack([b[:F], b[F:]]))

  h_t = ch0(x_t0, src_t, dstr_t, wp_t, Wt0, bias_t, gamma_t, beta_t)
  h_s = ch0(x_s0, src_s, dstr_s, wp_s, Ws0, bias_s, gamma_s, beta_s)

  tab1 = jnp.concatenate([h_t, h_s], axis=0)
  src12 = jnp.concatenate([src_t, src_s + N])
  dstr12 = jnp.concatenate([dstr_t, dstr_s], axis=0)
  w12 = jnp.concatenate([wp_t, wp_s])
  s2b = _sc_spmm(tab1, src12, dstr12, w12, per_core_edges=True)[:, :N, :]

  _, b1t = _combine_weights(Wt1)
  _, b1s = _combine_weights(Ws1)
  out_t = _tc_block(h_t, s2b, Wt1, bias_t[1], gamma_t[1], beta_t[1],
                    jnp.stack([b1t, zf]))
  out_s = _tc_block(h_s, s2b, Ws1, bias_s[1], gamma_s[1], beta_s[1],
                    jnp.stack([zf, b1s]))
  return (out_t, out_s)


# speedup vs baseline: 1.0173x; 1.0173x over previous
import functools

import jax
import jax.numpy as jnp
from jax import lax
from jax.experimental import pallas as pl
from jax.experimental.pallas import tpu as pltpu
from jax.experimental.pallas import tpu_sc as plsc

N = 50000
E = 800000
F = 32
EPS = 1e-5
SLOPE = 0.1

NCORE = 2
NSUB = 16
CH = 256
SB = 128
NSC = CH // SB
NBUF = 3
EP = 811008
NP = 50048
ROWS_PER_SUB = NP // NSUB


def _sc_spmm(table, src2, dst_r, w_p, per_core_edges):
  nchunks = (EP // NSUB) // CH
  ntrip = nchunks // NBUF
  mesh = plsc.VectorSubcoreMesh(core_axis_name="c", subcore_axis_name="s")
  zeros = jnp.zeros((NP, F), jnp.float32)

  @functools.partial(
      pl.kernel,
      out_type=jax.ShapeDtypeStruct((NCORE, NP, F), jnp.float32),
      mesh=mesh,
      compiler_params=pltpu.CompilerParams(use_tc_tiling_on_sc=False),
      scratch_types=[
          pltpu.VMEM((NBUF, CH), jnp.int32),
          pltpu.VMEM((NBUF, NSC, SB), jnp.int32),
          pltpu.VMEM((NBUF, CH), jnp.float32),
          pltpu.VMEM((NBUF, CH, F), jnp.float32),
          pltpu.VMEM_SHARED((NP, F), jnp.float32),
          pltpu.SemaphoreType.DMA((NBUF,)),
          pltpu.SemaphoreType.DMA((NBUF,)),
          pltpu.SemaphoreType.DMA((NBUF,)),
      ],
  )
  def spmm(table_h, src_h, dstr_h, w_h, zeros_h, out_h,
           srcv, dstv, wv, rows, acc, gsem, ssem, lsem):
    c = lax.axis_index("c")
    s = lax.axis_index("s")

    r0 = pl.multiple_of(s * ROWS_PER_SUB, 8)
    pltpu.sync_copy(zeros_h.at[pl.ds(r0, ROWS_PER_SUB)],
                    acc.at[pl.ds(r0, ROWS_PER_SUB)])
    plsc.subcore_barrier()

    base = s * (EP // NSUB)
    wbase = c * EP if per_core_edges else 0
    dbase = c * (EP // SB) if per_core_edges else 0

    def issue_linear_sw(k, b):
      e0 = pl.multiple_of(base + k * CH, CH)
      pltpu.async_copy(src_h.at[pl.ds(pl.multiple_of(c * EP + e0, CH), CH)],
                       srcv.at[b], lsem.at[b])
      pltpu.async_copy(w_h.at[pl.ds(pl.multiple_of(wbase + e0, CH), CH)],
                       wv.at[b], lsem.at[b])

    def issue_linear_d(k, b):
      e0 = pl.multiple_of(base + k * CH, CH)
      pltpu.async_copy(
          dstr_h.at[pl.ds(pl.multiple_of(dbase + e0 // SB, NSC), NSC)],
          dstv.at[b], lsem.at[b])

    def wait_linear(b):
      pltpu.make_async_copy(src_h.at[pl.ds(0, CH)], srcv.at[b],
                            lsem.at[b]).wait()
      pltpu.make_async_copy(w_h.at[pl.ds(0, CH)], wv.at[b],
                            lsem.at[b]).wait()
      pltpu.make_async_copy(dstr_h.at[pl.ds(0, NSC)], dstv.at[b],
                            lsem.at[b]).wait()

    def issue_gather(b):
      pltpu.async_copy(table_h.at[srcv.at[b]], rows.at[b], gsem.at[b])

    def wait_gather(b):
      pltpu.make_async_copy(zeros_h.at[pl.ds(0, CH)], rows.at[b],
                            gsem.at[b]).wait()

    def multiply(b):
      def group_body(g, _):
        w16 = wv[b, pl.ds(g * 16, 16)]
        for u in range(16):
          e = g * 16 + u
          ws = w16[u]
          for h in range(0, F, 16):
            rows[b, e, pl.ds(h, 16)] = rows[b, e, pl.ds(h, 16)] * ws
        return 0
      lax.fori_loop(0, CH // 16, group_body, 0)

    def issue_scatters(b):
      for j in range(NSC):
        pltpu.async_copy(rows.at[b, pl.ds(j * SB, SB)],
                         acc.at[dstv.at[b, j]], ssem.at[b], add=True)

    def drain_scatters(b):
      for j in range(NSC):
        pltpu.make_async_copy(zeros_h.at[pl.ds(0, SB)],
                              rows.at[b, pl.ds(j * SB, SB)],
                              ssem.at[b]).wait()

    for b in range(NBUF - 1):
      issue_linear_sw(b, b)
      issue_linear_d(b, b)
      wait_linear(b)
      issue_gather(b)

    def trip_body(t, _):
      for b in range(NBUF):
        k = t * NBUF + b
        bp = (b + NBUF - 1) % NBUF
        q = k + NBUF - 1

        if b == 0:
          issue_linear_sw(q, bp)
        else:
          @pl.when(t < ntrip - 1)
          def _():
            issue_linear_sw(q, bp)

        wait_gather(b)
        multiply(b)
        issue_scatters(b)

        def finish_refill():
          drain_scatters(bp)
          issue_linear_d(q, bp)
          wait_linear(bp)
          issue_gather(bp)

        if b == 0:
          @pl.when(t == 0)
          def _():
            issue_linear_d(q, bp)
            wait_linear(bp)
            issue_gather(bp)

          @pl.when(t > 0)
          def _():
            finish_refill()
        else:
          @pl.when(t < ntrip - 1)
          def _():
            finish_refill()

          @pl.when(t == ntrip - 1)
          def _():
            drain_scatters(bp)
      return 0

    lax.fori_loop(0, ntrip, trip_body, 0)
    drain_scatters((nchunks - 1) % NBUF)

    plsc.subcore_barrier()
    pltpu.sync_copy(acc.at[pl.ds(r0, ROWS_PER_SUB)],
                    out_h.at[c, pl.ds(r0, ROWS_PER_SUB)])

  return spmm(table, src2, dst_r, w_p, zeros)


BN = 2000


def _tc_conv_body(x_ref, s2_ref, a_ref, b2_ref, bias_ref, g_ref, be_ref,
                  o_ref, yv, s1v, sqv):
  p = pl.program_id(0)
  i = pl.program_id(1)

  @pl.when(p == 0)
  def _():
    y = jnp.dot(x_ref[...], a_ref[...], preferred_element_type=jnp.float32)
    y -= jnp.dot(s2_ref[0], b2_ref[0], preferred_element_type=jnp.float32)
    y -= jnp.dot(s2_ref[1], b2_ref[1], preferred_element_type=jnp.float32)
    y += bias_ref[...]
    yv[pl.ds(i * BN, BN)] = y
    part = jnp.sum(y.reshape(BN // 8, 8, F), axis=0)
    psq = jnp.sum((y * y).reshape(BN // 8, 8, F), axis=0)

    @pl.when(i == 0)
    def _():
      s1v[...] = part
      sqv[...] = psq

    @pl.when(i != 0)
    def _():
      s1v[...] += part
      sqv[...] += psq

  @pl.when(p == 1)
  def _():
    s1 = jnp.sum(s1v[...], axis=0, keepdims=True)
    sq = jnp.sum(sqv[...], axis=0, keepdims=True)
    mean = s1 / N
    var = sq / N - mean * mean
    scale = g_ref[...] * lax.rsqrt(var + EPS)
    shift = be_ref[...] - mean * scale
    o = yv[pl.ds(i * BN, BN)] * scale + shift
    o_ref[...] = jnp.where(o >= 0, o, SLOPE * o)


def _tc_conv(x, s2, a, b2, bias, gamma, beta):
  d = x.shape[1]
  return pl.pallas_call(
      _tc_conv_body,
      grid=(2, N // BN),
      in_specs=[
          pl.BlockSpec((BN, d), lambda p, i: ((1 - p) * i, 0)),
          pl.BlockSpec((2, BN, F), lambda p, i: (0, (1 - p) * i, 0)),
          pl.BlockSpec((d, F), lambda p, i: (0, 0)),
          pl.BlockSpec((2, F, F), lambda p, i: (0, 0, 0)),
          pl.BlockSpec((1, F), lambda p, i: (0, 0)),
          pl.BlockSpec((1, F), lambda p, i: (0, 0)),
          pl.BlockSpec((1, F), lambda p, i: (0, 0)),
      ],
      out_specs=pl.BlockSpec((BN, F), lambda p, i: (i, 0)),
      out_shape=jax.ShapeDtypeStruct((N, F), jnp.float32),
      scratch_shapes=[
          pltpu.VMEM((N, F), jnp.float32),
          pltpu.VMEM((8, F), jnp.float32),
          pltpu.VMEM((8, F), jnp.float32),
      ],
  )(x, s2, a, b2, bias, gamma, beta)


def _combine_weights(Ws):
  ks = jnp.arange(Ws.shape[0], dtype=jnp.float32)
  return jnp.sum(Ws, axis=0), jnp.einsum("k,kij->ij", ks, Ws)


def _tc_block(x, s2, Ws, bias, gamma, beta, b2):
  a, _ = _combine_weights(Ws)
  return _tc_conv(x, s2, a, b2, bias.reshape(1, F),
                  gamma.reshape(1, F), beta.reshape(1, F))


def _prep_edges(ei, w):
  pad = EP - E
  src = jnp.concatenate([ei[0].astype(jnp.int32), jnp.zeros((pad,), jnp.int32)])
  dst = jnp.concatenate([ei[1].astype(jnp.int32), jnp.zeros((pad,), jnp.int32)])
  w_p = jnp.concatenate([w, jnp.zeros((pad,), jnp.float32)])
  return src, dst.reshape(EP // SB, SB), w_p


def kernel(x_t0, edge_weight_t, x_s0, edge_weight_s, Wt0, Wt1, bias_t, gamma_t,
           beta_t, Ws0, Ws1, bias_s, gamma_s, beta_s, edge_index_t,
           edge_index_s):
  src_t, dstr_t, wp_t = _prep_edges(edge_index_t, edge_weight_t)
  src_s, dstr_s, wp_s = _prep_edges(edge_index_s, edge_weight_s)
  zf = jnp.zeros((F, F), jnp.float32)

  def ch0(x0, src, dstr, wp, Ws, bias, gamma, beta):
    table = jnp.concatenate([x0[:, :F], x0[:, F:]], axis=0)
    src2 = jnp.concatenate([src, src + N])
    s2 = _sc_spmm(table, src2, dstr, wp, per_core_edges=False)[:, :N, :]
    _, b = _combine_weights(Ws)
    return _tc_block(x0, s2, Ws, bias[0], gamma[0], beta[0],
                     jnp.stack([b[:F], b[F:]]))

  h_t = ch0(x_t0, src_t, dstr_t, wp_t, Wt0, bias_t, gamma_t, beta_t)
  h_s = ch0(x_s0, src_s, dstr_s, wp_s, Ws0, bias_s, gamma_s, beta_s)

  tab1 = jnp.concatenate([h_t, h_s], axis=0)
  src12 = jnp.concatenate([src_t, src_s + N])
  dstr12 = jnp.concatenate([dstr_t, dstr_s], axis=0)
  w12 = jnp.concatenate([wp_t, wp_s])
  s2b = _sc_spmm(tab1, src12, dstr12, w12, per_core_edges=True)[:, :N, :]

  _, b1t = _combine_weights(Wt1)
  _, b1s = _combine_weights(Ws1)
  out_t = _tc_block(h_t, s2b, Wt1, bias_t[1], gamma_t[1], beta_t[1],
                    jnp.stack([b1t, zf]))
  out_s = _tc_block(h_s, s2b, Ws1, bias_s[1], gamma_s[1], beta_s[1],
                    jnp.stack([zf, b1s]))
  return (out_t, out_s)
